# Initial kernel scaffold; baseline (speedup 1.0000x reference)
import jax
import jax.numpy as jnp
from jax.experimental import pallas as pl


def kernel(x, edge_index, W1, b1, W2, b2, a):
    raise NotImplementedError("write your pallas kernel here")


# speedup vs baseline: 13.4323x; 13.4323x over previous
import functools

import jax
import jax.numpy as jnp
from jax import lax
from jax.experimental import pallas as pl
from jax.experimental.pallas import tpu as pltpu
from jax.experimental.pallas import tpu_sc as plsc

_NC = 2
_NS = 16
_NW = _NC * _NS
_CHUNK = 128


def _sc_degree(dst2d, zeros_stripe, ones_chunk, n_pad, cpt):
    stripe = n_pad // _NS
    mesh = plsc.VectorSubcoreMesh(core_axis_name="c", subcore_axis_name="s")

    @functools.partial(
        pl.kernel,
        out_type=jax.ShapeDtypeStruct((_NC, n_pad), jnp.float32),
        mesh=mesh,
        scratch_types=[
            pltpu.VMEM((cpt, _CHUNK), jnp.int32),
            pltpu.VMEM((_CHUNK,), jnp.float32),
            pltpu.VMEM_SHARED((n_pad,), jnp.float32),
        ],
    )
    def deg_kernel(dst_hbm, z_hbm, ones_hbm, out_hbm, dst_v, ones_v, acc):
        cid = lax.axis_index("c")
        sid = lax.axis_index("s")
        wid = sid * _NC + cid
        pltpu.sync_copy(z_hbm, acc.at[pl.ds(sid * stripe, stripe)])
        pltpu.sync_copy(ones_hbm, ones_v)
        pltpu.sync_copy(dst_hbm.at[pl.ds(wid * cpt, cpt)], dst_v)
        plsc.subcore_barrier()

        def body(j, carry):
            pltpu.sync_copy(ones_v, acc.at[dst_v.at[j]], add=True)
            return carry

        lax.fori_loop(0, cpt, body, 0)
        plsc.subcore_barrier()
        pltpu.sync_copy(acc.at[pl.ds(sid * stripe, stripe)],
                        out_hbm.at[cid, pl.ds(sid * stripe, stripe)])

    return deg_kernel(dst2d, zeros_stripe, ones_chunk)


def _sc_scatter(tab, src2d, dst2d, zeros_rows, n_pad, cpt, width):
    stripe = n_pad // _NS
    mesh = plsc.VectorSubcoreMesh(core_axis_name="c", subcore_axis_name="s")

    @functools.partial(
        pl.kernel,
        out_type=jax.ShapeDtypeStruct((_NC, n_pad, width), jnp.float32),
        mesh=mesh,
        scratch_types=[
            pltpu.VMEM((cpt, _CHUNK), jnp.int32),
            pltpu.VMEM((cpt, _CHUNK), jnp.int32),
            pltpu.VMEM((_CHUNK, width), jnp.float32),
            pltpu.VMEM_SHARED((n_pad, width), jnp.float32),
            pltpu.SemaphoreType.DMA,
        ],
    )
    def scat_kernel(tab_hbm, src_hbm, dst_hbm, z_hbm, out_hbm,
                    src_v, dst_v, rows_v, acc, sem):
        cid = lax.axis_index("c")
        sid = lax.axis_index("s")
        wid = sid * _NC + cid
        pltpu.sync_copy(z_hbm, acc.at[pl.ds(sid * stripe, stripe)])
        pltpu.sync_copy(src_hbm.at[pl.ds(wid * cpt, cpt)], src_v)
        pltpu.sync_copy(dst_hbm.at[pl.ds(wid * cpt, cpt)], dst_v)
        plsc.subcore_barrier()

        def body(j, carry):
            pltpu.async_copy(tab_hbm.at[src_v.at[j]], rows_v, sem).wait()
            pltpu.sync_copy(rows_v, acc.at[dst_v.at[j]], add=True)
            return carry

        lax.fori_loop(0, cpt, body, 0)
        plsc.subcore_barrier()
        pltpu.sync_copy(acc.at[pl.ds(sid * stripe, stripe)],
                        out_hbm.at[cid, pl.ds(sid * stripe, stripe)])

    return scat_kernel(tab, src2d, dst2d, zeros_rows)


def _tc_prep(degp, x_pad, n_pad, width):
    blk = 1024

    def body(degp_ref, x_ref, d_ref, zs_ref):
        deg = degp_ref[0] + degp_ref[1] + 1.0
        d = lax.rsqrt(deg)
        d_ref[...] = d
        zs_ref[...] = x_ref[...] * d

    return pl.pallas_call(
        body,
        grid=(n_pad // blk,),
        in_specs=[
            pl.BlockSpec((2, blk, 1), lambda i: (0, i, 0)),
            pl.BlockSpec((blk, width), lambda i: (i, 0)),
        ],
        out_specs=[
            pl.BlockSpec((blk, 1), lambda i: (i, 0)),
            pl.BlockSpec((blk, width), lambda i: (i, 0)),
        ],
        out_shape=[
            jax.ShapeDtypeStruct((n_pad, 1), jnp.float32),
            jax.ShapeDtypeStruct((n_pad, width), jnp.float32),
        ],
    )(degp, x_pad)


def _tc_layer(P, zs, dcol, W1, b1r, W2, a11, n_pad):
    blk = 1024
    d_in, h1 = W1.shape
    h2 = W2.shape[1]

    def body(p_ref, zs_ref, d_ref, w1_ref, b1_ref, w2_ref, a_ref, out_ref):
        d = d_ref[...]
        agg = (p_ref[0] + p_ref[1] + zs_ref[...]) * d
        f1 = jnp.dot(agg, w1_ref[...], preferred_element_type=jnp.float32)
        f1 = f1 + b1_ref[...]
        aa = a_ref[0, 0]
        f1 = jnp.where(f1 >= 0, f1, aa * f1)
        hh = jnp.dot(f1, w2_ref[...], preferred_element_type=jnp.float32)
        out_ref[...] = hh * d

    return pl.pallas_call(
        body,
        grid=(n_pad // blk,),
        in_specs=[
            pl.BlockSpec((2, blk, d_in), lambda i: (0, i, 0)),
            pl.BlockSpec((blk, d_in), lambda i: (i, 0)),
            pl.BlockSpec((blk, 1), lambda i: (i, 0)),
            pl.BlockSpec((d_in, h1), lambda i: (0, 0)),
            pl.BlockSpec((1, h1), lambda i: (0, 0)),
            pl.BlockSpec((h1, h2), lambda i: (0, 0)),
            pl.BlockSpec(memory_space=pltpu.SMEM),
        ],
        out_specs=pl.BlockSpec((blk, h2), lambda i: (i, 0)),
        out_shape=jax.ShapeDtypeStruct((n_pad, h2), jnp.float32),
    )(P, zs, dcol, W1, b1r, W2, a11)


def _tc_final(Q, ys, dcol, b2r, a11, n_pad, width):
    blk = 1024

    def body(q_ref, ys_ref, d_ref, b2_ref, a_ref, out_ref):
        agg = (q_ref[0] + q_ref[1] + ys_ref[...]) * d_ref[...] + b2_ref[...]
        aa = a_ref[0, 0]
        out_ref[...] = jnp.where(agg >= 0, agg, aa * agg)

    return pl.pallas_call(
        body,
        grid=(n_pad // blk,),
        in_specs=[
            pl.BlockSpec((2, blk, width), lambda i: (0, i, 0)),
            pl.BlockSpec((blk, width), lambda i: (i, 0)),
            pl.BlockSpec((blk, 1), lambda i: (i, 0)),
            pl.BlockSpec((1, width), lambda i: (0, 0)),
            pl.BlockSpec(memory_space=pltpu.SMEM),
        ],
        out_specs=pl.BlockSpec((blk, width), lambda i: (i, 0)),
        out_shape=jax.ShapeDtypeStruct((n_pad, width), jnp.float32),
    )(Q, ys, dcol, b2r, a11)


def kernel(x, edge_index, W1, b1, W2, b2, a):
    n, d_in = x.shape
    e = edge_index.shape[1]
    n_pad = ((n + 1 + 2047) // 2048) * 2048
    cpt = -(-e // (_NW * _CHUNK))
    cpt = ((cpt + 7) // 8) * 8
    e_pad = _NW * cpt * _CHUNK

    src = edge_index[0].astype(jnp.int32)
    dst = edge_index[1].astype(jnp.int32)
    pad_idx = jnp.full((e_pad - e,), n, jnp.int32)
    src2d = jnp.concatenate([src, pad_idx]).reshape(_NW * cpt, _CHUNK)
    dst2d = jnp.concatenate([dst, pad_idx]).reshape(_NW * cpt, _CHUNK)

    x_pad = jnp.zeros((n_pad, d_in), jnp.float32).at[:n].set(x)
    zeros_stripe = jnp.zeros((n_pad // _NS,), jnp.float32)
    zeros_rows = jnp.zeros((n_pad // _NS, d_in), jnp.float32)
    ones_chunk = jnp.ones((_CHUNK,), jnp.float32)

    degp = _sc_degree(dst2d, zeros_stripe, ones_chunk, n_pad, cpt)
    dcol, zs = _tc_prep(degp.reshape(_NC, n_pad, 1), x_pad, n_pad, d_in)
    P = _sc_scatter(zs, src2d, dst2d, zeros_rows, n_pad, cpt, d_in)
    ys = _tc_layer(P, zs, dcol, W1, b1.reshape(1, -1), W2,
                   a.reshape(1, 1), n_pad)
    Q = _sc_scatter(ys, src2d, dst2d, zeros_rows, n_pad, cpt, W2.shape[1])
    out = _tc_final(Q, ys, dcol, b2.reshape(1, -1), a.reshape(1, 1),
                    n_pad, W2.shape[1])
    return out[:n]
